# baseline (speedup 1.0000x reference)
import jax
import jax.numpy as jnp
from jax.experimental import pallas as pl
from jax.experimental.pallas import tpu as pltpu

_ROWS = 4 * 4096
_D = 2048
_CH = 512
_NB = 8
_LEAD = 4
_NCHUNKS = _ROWS // _CH


def _dma_ring_body(x_ref, o_ref):
    def scoped(bufs, fsems, dsems):
        def fill(ci):
            s = ci % _NB
            return pltpu.make_async_copy(
                x_ref.at[pl.ds(ci * _CH, _CH)], bufs.at[s], fsems.at[s]
            )

        def drain(ci):
            s = ci % _NB
            return pltpu.make_async_copy(
                bufs.at[s], o_ref.at[pl.ds(ci * _CH, _CH)], dsems.at[s]
            )

        for i in range(_NCHUNKS + _LEAD):
            if i < _NCHUNKS:
                if i >= _NB:
                    drain(i - _NB).wait()
                fill(i).start()
            j = i - _LEAD
            if j >= 0:
                fill(j).wait()
                drain(j).start()
        for j in range(_NCHUNKS - _NB, _NCHUNKS):
            drain(j).wait()

    pl.run_scoped(
        scoped,
        pltpu.VMEM((_NB, _CH, _D), jnp.float32),
        pltpu.SemaphoreType.DMA((_NB,)),
        pltpu.SemaphoreType.DMA((_NB,)),
    )


def kernel(inputs, medians):
    del medians
    B, S, D = inputs.shape
    x = inputs.reshape(B * S, D)
    out = pl.pallas_call(
        _dma_ring_body,
        in_specs=[pl.BlockSpec(memory_space=pl.ANY)],
        out_specs=pl.BlockSpec(memory_space=pl.ANY),
        out_shape=jax.ShapeDtypeStruct((B * S, D), inputs.dtype),
    )(x)
    return out.reshape(B, S, D)
